# Initial kernel scaffold; baseline (speedup 1.0000x reference)
import jax
import jax.numpy as jnp
from jax.experimental import pallas as pl


def kernel(drug_features, cell_features_in_network, cell_features, drug_index, block0_d2c, block0_c2d, block1_d2c, block1_c2d, W_drug, b_drug, W_cell, b_cell, W_expr, b_expr, W1_d2c, b1_d2c, W1_c2d, b1_c2d, W2_c2d, b2_c2d, W_mid, b_mid, W_out, b_out):
    raise NotImplementedError("write your pallas kernel here")


# speedup vs baseline: 3.2876x; 3.2876x over previous
import functools

import jax
import jax.numpy as jnp
from jax import lax
from jax.experimental import pallas as pl
from jax.experimental.pallas import tpu as pltpu
from jax.experimental.pallas import tpu_sc as plsc

N_NODE = 10000
N_PAD = 10016
E = 38000
CD = 512
CHUNK = 128
NCHUNK = CD // CHUNK
NTILE = 16
EPT = E // NTILE
NB = 19
B = 1024
_ROWS = 624
_ZTAIL = N_PAD - NTILE * _ROWS
_DTAIL = N_NODE - NTILE * _ROWS


def _lrelu(x):
    return jnp.where(x >= 0, x, 0.01 * x)


def _mesh():
    return plsc.VectorSubcoreMesh(core_axis_name="c", subcore_axis_name="s")


def _zero_slices(zbuf, acc, s, tail):
    for p, sz in ((0, 128), (1, 128), (2, 128), (3, 128), (4, 112)):
        pltpu.sync_copy(zbuf.at[pl.ds(0, sz)],
                        acc.at[pl.ds(s * _ROWS + p * 128, sz)])

    @pl.when(s == NTILE - 1)
    def _():
        pltpu.sync_copy(zbuf.at[pl.ds(0, tail)],
                        acc.at[pl.ds(NTILE * _ROWS, tail)])


def _dump_slices(acc, out_slice_fn, s):
    pltpu.sync_copy(acc.at[pl.ds(s * _ROWS, _ROWS)],
                    out_slice_fn(s * _ROWS, _ROWS))

    @pl.when(s == NTILE - 1)
    def _():
        pltpu.sync_copy(acc.at[pl.ds(NTILE * _ROWS, _DTAIL)],
                        out_slice_fn(NTILE * _ROWS, _DTAIL))


def _fill_const(buf, n, val):
    def _row(i, _):
        for q in range(buf.shape[1] // 16):
            buf[i, pl.ds(q * 16, 16)] = jnp.full((16,), val, jnp.float32)
        return 0

    lax.fori_loop(0, n, _row, 0)


def _deg_body(hidx, z128, deg, a128, idx, o128):
    c = lax.axis_index("c")
    s = lax.axis_index("s")
    _fill_const(o128, 128, 1.0)

    def _round(h, out_idx):
        _zero_slices(z128, a128, s, _ZTAIL)
        plsc.subcore_barrier()
        pltpu.sync_copy(hidx.at[h, s], idx)
        for j in range(NB):
            pltpu.sync_copy(o128, a128.at[idx.at[j]], add=True)
        plsc.subcore_barrier()
        _dump_slices(a128, lambda o, n: deg.at[out_idx, pl.ds(o, n)], s)
        plsc.subcore_barrier()

    _round(3 * c, 3 * c)
    _round(1 + 4 * c, 1 + 4 * c)
    _round(2 + 2 * c, 2 + 2 * c)


def _degrees(hidx, z128):
    k = pl.kernel(
        _deg_body,
        out_type=jax.ShapeDtypeStruct((6, N_NODE, 128), jnp.float32),
        mesh=_mesh(),
        scratch_types=[
            pltpu.VMEM_SHARED((N_PAD, 128), jnp.float32),
            pltpu.VMEM((NB, 128), jnp.int32),
            pltpu.VMEM((128, 128), jnp.float32),
        ],
    )
    return k(hidx, z128)


def _segsum_body(tflat, src_g, dst_s, zbuf, out, acc, idxs, idxd, rows, sem):
    c = lax.axis_index("c")
    s = lax.axis_index("s")
    pltpu.sync_copy(dst_s.at[s], idxd)

    for cc in range(2):
        c2 = 2 * c + cc
        for p, sz in ((0, 128), (1, 128), (2, 128), (3, 128), (4, 112)):
            pltpu.sync_copy(zbuf.at[pl.ds(0, sz)],
                            acc.at[pl.ds(s * _ROWS + p * 128, sz)])

        @pl.when(s == NTILE - 1)
        def _():
            pltpu.sync_copy(zbuf.at[pl.ds(0, _ZTAIL)],
                            acc.at[pl.ds(NTILE * _ROWS, _ZTAIL)])

        plsc.subcore_barrier()
        pltpu.sync_copy(src_g.at[c2, s], idxs)
        for j in range(NB):
            pltpu.async_copy(tflat.at[idxs.at[j]], rows, sem).wait()
            pltpu.sync_copy(rows, acc.at[idxd.at[j]], add=True)
        plsc.subcore_barrier()
        pltpu.sync_copy(acc.at[pl.ds(s * _ROWS, _ROWS)],
                        out.at[c2, pl.ds(s * _ROWS, _ROWS)])

        @pl.when(s == NTILE - 1)
        def _():
            pltpu.sync_copy(acc.at[pl.ds(NTILE * _ROWS, _DTAIL)],
                            out.at[c2, pl.ds(NTILE * _ROWS, _DTAIL)])

        plsc.subcore_barrier()


def _segsum(tflat, src_g, dst_s, z128):
    k = pl.kernel(
        _segsum_body,
        out_type=jax.ShapeDtypeStruct((NCHUNK, N_NODE, CHUNK), jnp.float32),
        mesh=_mesh(),
        scratch_types=[
            pltpu.VMEM_SHARED((N_PAD, CHUNK), jnp.float32),
            pltpu.VMEM((NB, 128), jnp.int32),
            pltpu.VMEM((NB, 128), jnp.int32),
            pltpu.VMEM((128, CHUNK), jnp.float32),
            pltpu.SemaphoreType.DMA,
        ],
    )
    return k(tflat, src_g, dst_s, z128)


def _gather1_body(mflat, denc, deg1t, deg5t, dix, m_sel, enc_sel, d1_sel,
                  d5_sel, idxv, tmp, r128, r512, sem):
    c = lax.axis_index("c")
    s = lax.axis_index("s")
    base = (s * 2 + c) * 32
    pltpu.sync_copy(dix.at[pl.ds(base, 32)], idxv)
    pltpu.async_copy(denc.at[idxv], r512, sem).wait()
    pltpu.sync_copy(r512, enc_sel.at[pl.ds(base, 32)])
    pltpu.async_copy(deg1t.at[idxv], r128, sem).wait()
    pltpu.sync_copy(r128, d1_sel.at[pl.ds(base, 32)])
    pltpu.async_copy(deg5t.at[idxv], r128, sem).wait()
    pltpu.sync_copy(r128, d5_sel.at[pl.ds(base, 32)])
    for ch in range(NCHUNK):
        for q in range(2):
            tmp[pl.ds(q * 16, 16)] = idxv[pl.ds(q * 16, 16)] + ch * N_NODE
        pltpu.async_copy(mflat.at[tmp], r128, sem).wait()
        pltpu.sync_copy(r128, m_sel.at[ch, pl.ds(base, 32)])


def _gather1(mflat, denc, deg1t, deg5t, dix):
    k = pl.kernel(
        _gather1_body,
        out_type=(
            jax.ShapeDtypeStruct((NCHUNK, B, CHUNK), jnp.float32),
            jax.ShapeDtypeStruct((B, CD), jnp.float32),
            jax.ShapeDtypeStruct((B, 128), jnp.float32),
            jax.ShapeDtypeStruct((B, 128), jnp.float32),
        ),
        mesh=_mesh(),
        scratch_types=[
            pltpu.VMEM((32,), jnp.int32),
            pltpu.VMEM((32,), jnp.int32),
            pltpu.VMEM((32, CHUNK), jnp.float32),
            pltpu.VMEM((32, CD), jnp.float32),
            pltpu.SemaphoreType.DMA,
        ],
    )
    return k(mflat, denc, deg1t, deg5t, dix)


def _gather2_body(mflat, dix, m_sel, idxv, tmp, r128, sem):
    c = lax.axis_index("c")
    s = lax.axis_index("s")
    base = (s * 2 + c) * 32
    pltpu.sync_copy(dix.at[pl.ds(base, 32)], idxv)
    for ch in range(NCHUNK):
        for q in range(2):
            tmp[pl.ds(q * 16, 16)] = idxv[pl.ds(q * 16, 16)] + ch * N_NODE
        pltpu.async_copy(mflat.at[tmp], r128, sem).wait()
        pltpu.sync_copy(r128, m_sel.at[ch, pl.ds(base, 32)])


def _gather2(mflat, dix):
    k = pl.kernel(
        _gather2_body,
        out_type=jax.ShapeDtypeStruct((NCHUNK, B, CHUNK), jnp.float32),
        mesh=_mesh(),
        scratch_types=[
            pltpu.VMEM((32,), jnp.int32),
            pltpu.VMEM((32,), jnp.int32),
            pltpu.VMEM((32, CHUNK), jnp.float32),
            pltpu.SemaphoreType.DMA,
        ],
    )
    return k(mflat, dix)


def _enc_body(x, w, b, deg, enc, tout):
    a = jnp.dot(x[...], w[...], preferred_element_type=jnp.float32)
    e = _lrelu(a + b[...])
    enc[...] = e
    sc = lax.rsqrt(jnp.maximum(deg[:, 0:1], 1.0))
    t = e * sc
    for ch in range(NCHUNK):
        tout[ch] = t[:, ch * CHUNK:(ch + 1) * CHUNK]


def _encoder(x, w, b, deg, bm):
    n, kdim = x.shape
    grid = (n // bm,)
    return pl.pallas_call(
        _enc_body,
        grid=grid,
        in_specs=[
            pl.BlockSpec((bm, kdim), lambda m: (m, 0)),
            pl.BlockSpec((kdim, CD), lambda m: (0, 0)),
            pl.BlockSpec((1, CD), lambda m: (0, 0)),
            pl.BlockSpec((bm, 128), lambda m: (m, 0)),
        ],
        out_specs=[
            pl.BlockSpec((bm, CD), lambda m: (m, 0)),
            pl.BlockSpec((NCHUNK, bm, CHUNK), lambda m: (0, m, 0)),
        ],
        out_shape=[
            jax.ShapeDtypeStruct((n, CD), jnp.float32),
            jax.ShapeDtypeStruct((NCHUNK, n, CHUNK), jnp.float32),
        ],
    )(x, w, b, deg)


def _h1cell_body(m, w, b, enc, dgi, dgo, tout, acc):
    k = pl.program_id(1)

    @pl.when(k == 0)
    def _():
        acc[...] = jnp.zeros_like(acc)

    acc[...] += jnp.dot(m[0], w[...], preferred_element_type=jnp.float32)

    @pl.when(k == NCHUNK - 1)
    def _():
        si = lax.rsqrt(jnp.maximum(dgi[:, 0:1], 1.0))
        so = lax.rsqrt(jnp.maximum(dgo[:, 0:1], 1.0))
        h = _lrelu(acc[...] * si + b[...] + 0.5 * enc[...]) * so
        for ch in range(NCHUNK):
            tout[ch] = h[:, ch * CHUNK:(ch + 1) * CHUNK]


def _h1cell(m_cell, w, b, enc, dgi, dgo, bm):
    n = enc.shape[0]
    grid = (n // bm, NCHUNK)
    return pl.pallas_call(
        _h1cell_body,
        grid=grid,
        in_specs=[
            pl.BlockSpec((1, bm, CHUNK), lambda m, k: (k, m, 0)),
            pl.BlockSpec((CHUNK, CD), lambda m, k: (k, 0)),
            pl.BlockSpec((1, CD), lambda m, k: (0, 0)),
            pl.BlockSpec((bm, CD), lambda m, k: (m, 0)),
            pl.BlockSpec((bm, 128), lambda m, k: (m, 0)),
            pl.BlockSpec((bm, 128), lambda m, k: (m, 0)),
        ],
        out_specs=pl.BlockSpec((NCHUNK, bm, CHUNK), lambda m, k: (0, m, 0)),
        out_shape=jax.ShapeDtypeStruct((NCHUNK, n, CHUNK), jnp.float32),
        scratch_shapes=[pltpu.VMEM((bm, CD), jnp.float32)],
    )(m_cell, w, b, enc, dgi, dgo)


def _expr_body(x, w, b, out):
    out[...] = _lrelu(
        jnp.dot(x[...], w[...], preferred_element_type=jnp.float32) + b[...])


def _expr(x, w, b, bm):
    n, kdim = x.shape
    ee = w.shape[1]
    return pl.pallas_call(
        _expr_body,
        grid=(n // bm,),
        in_specs=[
            pl.BlockSpec((bm, kdim), lambda m: (m, 0)),
            pl.BlockSpec((kdim, ee), lambda m: (0, 0)),
            pl.BlockSpec((1, ee), lambda m: (0, 0)),
        ],
        out_specs=pl.BlockSpec((bm, ee), lambda m: (m, 0)),
        out_shape=jax.ShapeDtypeStruct((n, ee), jnp.float32),
    )(x, w, b)


def _head_body(expr, m1, enc1, d1, d5, m2, w1, b1, w2, b2, wm, bm_, wo, bo,
               out):
    p1 = jnp.zeros((B, CD), jnp.float32)
    p2 = jnp.zeros((B, CD), jnp.float32)
    for ch in range(NCHUNK):
        wch = w1[pl.ds(ch * CHUNK, CHUNK), :]
        p1 = p1 + jnp.dot(m1[ch], wch, preferred_element_type=jnp.float32)
        wch2 = w2[pl.ds(ch * CHUNK, CHUNK), :]
        p2 = p2 + jnp.dot(m2[ch], wch2, preferred_element_type=jnp.float32)
    s1 = lax.rsqrt(jnp.maximum(d1[:, 0:1], 1.0))
    s5 = lax.rsqrt(jnp.maximum(d5[:, 0:1], 1.0))
    h1 = _lrelu(p1 * s1 + b1[...] + 0.5 * enc1[...])
    h2 = _lrelu(p2 * s5 + b2[...] + 0.5 * h1)
    ee = expr.shape[1]
    mid = _lrelu(
        jnp.dot(expr[...], wm[pl.ds(0, ee), :],
                preferred_element_type=jnp.float32)
        + jnp.dot(h2, wm[pl.ds(ee, CD), :],
                  preferred_element_type=jnp.float32)
        + bm_[...])
    out[...] = jnp.dot(mid, wo[...], preferred_element_type=jnp.float32) \
        + bo[...]


def _head(expr, m1, enc1, d1, d5, m2, w1, b1, w2, b2, wm, bmid, wo, bo):
    ee = expr.shape[1]
    mids = wm.shape[1]
    full = lambda *shape: pl.BlockSpec(shape, lambda: tuple(0 for _ in shape))
    return pl.pallas_call(
        _head_body,
        grid=(),
        in_specs=[
            full(B, ee),
            full(NCHUNK, B, CHUNK),
            full(B, CD),
            full(B, 128),
            full(B, 128),
            full(NCHUNK, B, CHUNK),
            full(CD, CD),
            full(1, CD),
            full(CD, CD),
            full(1, CD),
            full(ee + CD, mids),
            full(1, mids),
            full(mids, 1),
            full(1, 1),
        ],
        out_specs=full(B, 1),
        out_shape=jax.ShapeDtypeStruct((B, 1), jnp.float32),
    )(expr, m1, enc1, d1, d5, m2, w1, b1, w2, b2, wm, bmid, wo, bo)


def _pad_idx(a, fill):
    a = a.reshape(NTILE, EPT)
    a = jnp.pad(a, ((0, 0), (0, NB * 128 - EPT)), constant_values=fill)
    return a.reshape(NTILE, NB, 128)


def _src_gather_idx(src):
    base = _pad_idx(src, 0)
    offs = (jnp.arange(NCHUNK, dtype=jnp.int32) * N_NODE)[:, None, None, None]
    return base[None] + offs


def kernel(drug_features, cell_features_in_network, cell_features, drug_index,
           block0_d2c, block0_c2d, block1_d2c, block1_c2d, W_drug, b_drug,
           W_cell, b_cell, W_expr, b_expr, W1_d2c, b1_d2c, W1_c2d, b1_c2d,
           W2_c2d, b2_c2d, W_mid, b_mid, W_out, b_out):
    del block1_d2c

    hidx = jnp.stack([
        _pad_idx(block0_c2d[0], N_NODE),
        _pad_idx(block0_c2d[1], N_NODE),
        _pad_idx(block0_d2c[0], N_NODE),
        _pad_idx(block0_d2c[1], N_NODE),
        _pad_idx(block1_c2d[0], N_NODE),
        _pad_idx(block1_c2d[1], N_NODE),
    ])
    sg_c2d = _src_gather_idx(block0_c2d[0])
    ds_c2d = _pad_idx(block0_c2d[1], N_NODE)
    sg_d2c = _src_gather_idx(block0_d2c[0])
    ds_d2c = _pad_idx(block0_d2c[1], N_NODE)
    sg_b1 = _src_gather_idx(block1_c2d[0])
    ds_b1 = _pad_idx(block1_c2d[1], N_NODE)

    b_drug2 = b_drug.reshape(1, CD)
    b_cell2 = b_cell.reshape(1, CD)
    b_expr2 = b_expr.reshape(1, -1)
    b1_d2c2 = b1_d2c.reshape(1, CD)
    b1_c2d2 = b1_c2d.reshape(1, CD)
    b2_c2d2 = b2_c2d.reshape(1, CD)
    b_mid2 = b_mid.reshape(1, -1)
    b_out2 = b_out.reshape(1, 1)

    z128 = jnp.zeros((128, 128), jnp.float32)

    deg = _degrees(hidx, z128)
    d0, d1t, d2, d3, d4, d5t = (deg[i] for i in range(6))

    cell_enc, t_cell = _encoder(cell_features_in_network, W_cell, b_cell2,
                                d0, 1000)
    drug_enc, t_drug = _encoder(drug_features, W_drug, b_drug2, d2, 1000)

    m_cell = _segsum(t_drug.reshape(NCHUNK * N_NODE, CHUNK), sg_d2c, ds_d2c,
                     z128)
    m_drug = _segsum(t_cell.reshape(NCHUNK * N_NODE, CHUNK), sg_c2d, ds_c2d,
                     z128)

    m1_sel, enc_sel, d1_sel, d5_sel = _gather1(
        m_drug.reshape(NCHUNK * N_NODE, CHUNK), drug_enc, d1t, d5t,
        drug_index)

    t_h1 = _h1cell(m_cell, W1_d2c, b1_d2c2, cell_enc, d3, d4, 1000)

    m2 = _segsum(t_h1.reshape(NCHUNK * N_NODE, CHUNK), sg_b1, ds_b1, z128)
    m2_sel = _gather2(m2.reshape(NCHUNK * N_NODE, CHUNK), drug_index)

    expr_enc = _expr(cell_features, W_expr, b_expr2, 512)
    out = _head(expr_enc, m1_sel, enc_sel, d1_sel, d5_sel, m2_sel,
                W1_c2d, b1_c2d2, W2_c2d, b2_c2d2, W_mid, b_mid2, W_out,
                b_out2)
    return out


# speedup vs baseline: 3.6985x; 1.1250x over previous
import functools

import jax
import jax.numpy as jnp
from jax import lax
from jax.experimental import pallas as pl
from jax.experimental.pallas import tpu as pltpu
from jax.experimental.pallas import tpu_sc as plsc

N_NODE = 10000
N_PAD = 10016
E = 38000
CD = 512
CHUNK = 128
NCHUNK = CD // CHUNK
NTILE = 16
EPT = E // NTILE
NB = 19
B = 1024
_ROWS = 624
_ZTAIL = N_PAD - NTILE * _ROWS
_DTAIL = N_NODE - NTILE * _ROWS


def _lrelu(x):
    return jnp.where(x >= 0, x, 0.01 * x)


def _mesh():
    return plsc.VectorSubcoreMesh(core_axis_name="c", subcore_axis_name="s")


def _zero_slices(zbuf, acc, s, tail):
    for p, sz in ((0, 128), (1, 128), (2, 128), (3, 128), (4, 112)):
        pltpu.sync_copy(zbuf.at[pl.ds(0, sz)],
                        acc.at[pl.ds(s * _ROWS + p * 128, sz)])

    @pl.when(s == NTILE - 1)
    def _():
        pltpu.sync_copy(zbuf.at[pl.ds(0, tail)],
                        acc.at[pl.ds(NTILE * _ROWS, tail)])


def _dump_slices(acc, out_slice_fn, s):
    pltpu.sync_copy(acc.at[pl.ds(s * _ROWS, _ROWS)],
                    out_slice_fn(s * _ROWS, _ROWS))

    @pl.when(s == NTILE - 1)
    def _():
        pltpu.sync_copy(acc.at[pl.ds(NTILE * _ROWS, _DTAIL)],
                        out_slice_fn(NTILE * _ROWS, _DTAIL))


def _fill_const(buf, n, val):
    def _row(i, _):
        for q in range(buf.shape[1] // 16):
            buf[i, pl.ds(q * 16, 16)] = jnp.full((16,), val, jnp.float32)
        return 0

    lax.fori_loop(0, n, _row, 0)


def _deg_body(hidx, z128, deg, a128, idx, o128, sem):
    c = lax.axis_index("c")
    s = lax.axis_index("s")
    _fill_const(o128, 128, 1.0)

    def _round(h, out_idx):
        _zero_slices(z128, a128, s, _ZTAIL)
        plsc.subcore_barrier()
        pltpu.sync_copy(hidx.at[h, s], idx)
        descs = [pltpu.async_copy(o128, a128.at[idx.at[j]], sem, add=True)
                 for j in range(NB)]
        for dsc in descs:
            dsc.wait()
        plsc.subcore_barrier()
        _dump_slices(a128, lambda o, n: deg.at[out_idx, pl.ds(o, n)], s)
        plsc.subcore_barrier()

    _round(3 * c, 3 * c)
    _round(1 + 4 * c, 1 + 4 * c)
    _round(2 + 2 * c, 2 + 2 * c)


def _degrees(hidx, z128):
    k = pl.kernel(
        _deg_body,
        out_type=jax.ShapeDtypeStruct((6, N_NODE, 128), jnp.float32),
        mesh=_mesh(),
        scratch_types=[
            pltpu.VMEM_SHARED((N_PAD, 128), jnp.float32),
            pltpu.VMEM((NB, 128), jnp.int32),
            pltpu.VMEM((128, 128), jnp.float32),
            pltpu.SemaphoreType.DMA,
        ],
    )
    return k(hidx, z128)


def _segsum_body(tflat, src_g, dst_s, zbuf, out, acc, idxs, idxd, r0, r1,
                 g0, g1, s0, s1):
    c = lax.axis_index("c")
    s = lax.axis_index("s")
    pltpu.sync_copy(dst_s.at[s], idxd)
    bufs, gsems, ssems = (r0, r1), (g0, g1), (s0, s1)

    for cc in range(2):
        c2 = 2 * c + cc
        _zero_slices(zbuf, acc, s, _ZTAIL)
        plsc.subcore_barrier()
        pltpu.sync_copy(src_g.at[c2, s], idxs)
        gd = [None, None]
        sd = [None, None]
        for j in range(NB + 1):
            if j < NB:
                b = j % 2
                if sd[b] is not None:
                    sd[b].wait()
                gd[b] = pltpu.async_copy(tflat.at[idxs.at[j]], bufs[b],
                                         gsems[b])
            if j >= 1:
                bb = (j - 1) % 2
                gd[bb].wait()
                sd[bb] = pltpu.async_copy(bufs[bb], acc.at[idxd.at[j - 1]],
                                          ssems[bb], add=True)
        sd[(NB - 1) % 2].wait()
        sd[NB % 2].wait()
        plsc.subcore_barrier()
        _dump_slices(acc, lambda o, n: out.at[c2, pl.ds(o, n)], s)
        plsc.subcore_barrier()


def _segsum(tflat, src_g, dst_s, z128):
    k = pl.kernel(
        _segsum_body,
        out_type=jax.ShapeDtypeStruct((NCHUNK, N_NODE, CHUNK), jnp.float32),
        mesh=_mesh(),
        scratch_types=[
            pltpu.VMEM_SHARED((N_PAD, CHUNK), jnp.float32),
            pltpu.VMEM((NB, 128), jnp.int32),
            pltpu.VMEM((NB, 128), jnp.int32),
            pltpu.VMEM((128, CHUNK), jnp.float32),
            pltpu.VMEM((128, CHUNK), jnp.float32),
            pltpu.SemaphoreType.DMA,
            pltpu.SemaphoreType.DMA,
            pltpu.SemaphoreType.DMA,
            pltpu.SemaphoreType.DMA,
        ],
    )
    return k(tflat, src_g, dst_s, z128)


def _gather1_body(mflat, denc, deg1t, deg5t, dix, m_sel, enc_sel, d1_sel,
                  d5_sel, idxv, tmp, r128, r512, sem):
    c = lax.axis_index("c")
    s = lax.axis_index("s")
    base = (s * 2 + c) * 32
    pltpu.sync_copy(dix.at[pl.ds(base, 32)], idxv)
    pltpu.async_copy(denc.at[idxv], r512, sem).wait()
    pltpu.sync_copy(r512, enc_sel.at[pl.ds(base, 32)])
    pltpu.async_copy(deg1t.at[idxv], r128, sem).wait()
    pltpu.sync_copy(r128, d1_sel.at[pl.ds(base, 32)])
    pltpu.async_copy(deg5t.at[idxv], r128, sem).wait()
    pltpu.sync_copy(r128, d5_sel.at[pl.ds(base, 32)])
    for ch in range(NCHUNK):
        for q in range(2):
            tmp[pl.ds(q * 16, 16)] = idxv[pl.ds(q * 16, 16)] + ch * N_NODE
        pltpu.async_copy(mflat.at[tmp], r128, sem).wait()
        pltpu.sync_copy(r128, m_sel.at[ch, pl.ds(base, 32)])


def _gather1(mflat, denc, deg1t, deg5t, dix):
    k = pl.kernel(
        _gather1_body,
        out_type=(
            jax.ShapeDtypeStruct((NCHUNK, B, CHUNK), jnp.float32),
            jax.ShapeDtypeStruct((B, CD), jnp.float32),
            jax.ShapeDtypeStruct((B, 128), jnp.float32),
            jax.ShapeDtypeStruct((B, 128), jnp.float32),
        ),
        mesh=_mesh(),
        scratch_types=[
            pltpu.VMEM((32,), jnp.int32),
            pltpu.VMEM((32,), jnp.int32),
            pltpu.VMEM((32, CHUNK), jnp.float32),
            pltpu.VMEM((32, CD), jnp.float32),
            pltpu.SemaphoreType.DMA,
        ],
    )
    return k(mflat, denc, deg1t, deg5t, dix)


def _gather2_body(mflat, dix, m_sel, idxv, tmp, r128, sem):
    c = lax.axis_index("c")
    s = lax.axis_index("s")
    base = (s * 2 + c) * 32
    pltpu.sync_copy(dix.at[pl.ds(base, 32)], idxv)
    for ch in range(NCHUNK):
        for q in range(2):
            tmp[pl.ds(q * 16, 16)] = idxv[pl.ds(q * 16, 16)] + ch * N_NODE
        pltpu.async_copy(mflat.at[tmp], r128, sem).wait()
        pltpu.sync_copy(r128, m_sel.at[ch, pl.ds(base, 32)])


def _gather2(mflat, dix):
    k = pl.kernel(
        _gather2_body,
        out_type=jax.ShapeDtypeStruct((NCHUNK, B, CHUNK), jnp.float32),
        mesh=_mesh(),
        scratch_types=[
            pltpu.VMEM((32,), jnp.int32),
            pltpu.VMEM((32,), jnp.int32),
            pltpu.VMEM((32, CHUNK), jnp.float32),
            pltpu.SemaphoreType.DMA,
        ],
    )
    return k(mflat, dix)


def _enc_body(x, w, b, enc):
    a = jnp.dot(x[...], w[...], preferred_element_type=jnp.float32)
    enc[...] = _lrelu(a + b[...])


def _encoder(x, w, b, bm):
    n, kdim = x.shape
    return pl.pallas_call(
        _enc_body,
        grid=(n // bm,),
        in_specs=[
            pl.BlockSpec((bm, kdim), lambda m: (m, 0)),
            pl.BlockSpec((kdim, CD), lambda m: (0, 0)),
            pl.BlockSpec((1, CD), lambda m: (0, 0)),
        ],
        out_specs=pl.BlockSpec((bm, CD), lambda m: (m, 0)),
        out_shape=jax.ShapeDtypeStruct((n, CD), jnp.float32),
    )(x, w, b)


def _scale_body(enc, deg, tout):
    sc = lax.rsqrt(jnp.maximum(deg[:, 0:1], 1.0))
    t = enc[...] * sc
    for ch in range(NCHUNK):
        tout[ch] = t[:, ch * CHUNK:(ch + 1) * CHUNK]


def _scale(enc, deg, bm):
    n = enc.shape[0]
    return pl.pallas_call(
        _scale_body,
        grid=(n // bm,),
        in_specs=[
            pl.BlockSpec((bm, CD), lambda m: (m, 0)),
            pl.BlockSpec((bm, 128), lambda m: (m, 0)),
        ],
        out_specs=pl.BlockSpec((NCHUNK, bm, CHUNK), lambda m: (0, m, 0)),
        out_shape=jax.ShapeDtypeStruct((NCHUNK, n, CHUNK), jnp.float32),
    )(enc, deg)


def _h1cell_body(m, w, b, enc, dgi, dgo, tout, acc):
    k = pl.program_id(1)

    @pl.when(k == 0)
    def _():
        acc[...] = jnp.zeros_like(acc)

    acc[...] += jnp.dot(m[0], w[...], preferred_element_type=jnp.float32)

    @pl.when(k == NCHUNK - 1)
    def _():
        si = lax.rsqrt(jnp.maximum(dgi[:, 0:1], 1.0))
        so = lax.rsqrt(jnp.maximum(dgo[:, 0:1], 1.0))
        h = _lrelu(acc[...] * si + b[...] + 0.5 * enc[...]) * so
        for ch in range(NCHUNK):
            tout[ch] = h[:, ch * CHUNK:(ch + 1) * CHUNK]


def _h1cell(m_cell, w, b, enc, dgi, dgo, bm):
    n = enc.shape[0]
    grid = (n // bm, NCHUNK)
    return pl.pallas_call(
        _h1cell_body,
        grid=grid,
        in_specs=[
            pl.BlockSpec((1, bm, CHUNK), lambda m, k: (k, m, 0)),
            pl.BlockSpec((CHUNK, CD), lambda m, k: (k, 0)),
            pl.BlockSpec((1, CD), lambda m, k: (0, 0)),
            pl.BlockSpec((bm, CD), lambda m, k: (m, 0)),
            pl.BlockSpec((bm, 128), lambda m, k: (m, 0)),
            pl.BlockSpec((bm, 128), lambda m, k: (m, 0)),
        ],
        out_specs=pl.BlockSpec((NCHUNK, bm, CHUNK), lambda m, k: (0, m, 0)),
        out_shape=jax.ShapeDtypeStruct((NCHUNK, n, CHUNK), jnp.float32),
        scratch_shapes=[pltpu.VMEM((bm, CD), jnp.float32)],
    )(m_cell, w, b, enc, dgi, dgo)


def _expr_body(x, w, b, out):
    out[...] = _lrelu(
        jnp.dot(x[...], w[...], preferred_element_type=jnp.float32) + b[...])


def _expr(x, w, b, bm):
    n, kdim = x.shape
    ee = w.shape[1]
    return pl.pallas_call(
        _expr_body,
        grid=(n // bm,),
        in_specs=[
            pl.BlockSpec((bm, kdim), lambda m: (m, 0)),
            pl.BlockSpec((kdim, ee), lambda m: (0, 0)),
            pl.BlockSpec((1, ee), lambda m: (0, 0)),
        ],
        out_specs=pl.BlockSpec((bm, ee), lambda m: (m, 0)),
        out_shape=jax.ShapeDtypeStruct((n, ee), jnp.float32),
    )(x, w, b)


def _head_body(expr, m1, enc1, d1, d5, m2, w1, b1, w2, b2, wm, bm_, wo, bo,
               out):
    p1 = jnp.zeros((B, CD), jnp.float32)
    p2 = jnp.zeros((B, CD), jnp.float32)
    for ch in range(NCHUNK):
        wch = w1[pl.ds(ch * CHUNK, CHUNK), :]
        p1 = p1 + jnp.dot(m1[ch], wch, preferred_element_type=jnp.float32)
        wch2 = w2[pl.ds(ch * CHUNK, CHUNK), :]
        p2 = p2 + jnp.dot(m2[ch], wch2, preferred_element_type=jnp.float32)
    s1 = lax.rsqrt(jnp.maximum(d1[:, 0:1], 1.0))
    s5 = lax.rsqrt(jnp.maximum(d5[:, 0:1], 1.0))
    h1 = _lrelu(p1 * s1 + b1[...] + 0.5 * enc1[...])
    h2 = _lrelu(p2 * s5 + b2[...] + 0.5 * h1)
    ee = expr.shape[1]
    mid = _lrelu(
        jnp.dot(expr[...], wm[pl.ds(0, ee), :],
                preferred_element_type=jnp.float32)
        + jnp.dot(h2, wm[pl.ds(ee, CD), :],
                  preferred_element_type=jnp.float32)
        + bm_[...])
    out[...] = jnp.dot(mid, wo[...], preferred_element_type=jnp.float32) \
        + bo[...]


def _head(expr, m1, enc1, d1, d5, m2, w1, b1, w2, b2, wm, bmid, wo, bo):
    ee = expr.shape[1]
    mids = wm.shape[1]
    full = lambda *shape: pl.BlockSpec(shape, lambda: tuple(0 for _ in shape))
    return pl.pallas_call(
        _head_body,
        grid=(),
        in_specs=[
            full(B, ee),
            full(NCHUNK, B, CHUNK),
            full(B, CD),
            full(B, 128),
            full(B, 128),
            full(NCHUNK, B, CHUNK),
            full(CD, CD),
            full(1, CD),
            full(CD, CD),
            full(1, CD),
            full(ee + CD, mids),
            full(1, mids),
            full(mids, 1),
            full(1, 1),
        ],
        out_specs=full(B, 1),
        out_shape=jax.ShapeDtypeStruct((B, 1), jnp.float32),
    )(expr, m1, enc1, d1, d5, m2, w1, b1, w2, b2, wm, bmid, wo, bo)


def _pad_idx(a, fill):
    a = a.reshape(NTILE, EPT)
    a = jnp.pad(a, ((0, 0), (0, NB * 128 - EPT)), constant_values=fill)
    return a.reshape(NTILE, NB, 128)


def _src_gather_idx(src):
    base = _pad_idx(src, 0)
    offs = (jnp.arange(NCHUNK, dtype=jnp.int32) * N_NODE)[:, None, None, None]
    return base[None] + offs


def kernel(drug_features, cell_features_in_network, cell_features, drug_index,
           block0_d2c, block0_c2d, block1_d2c, block1_c2d, W_drug, b_drug,
           W_cell, b_cell, W_expr, b_expr, W1_d2c, b1_d2c, W1_c2d, b1_c2d,
           W2_c2d, b2_c2d, W_mid, b_mid, W_out, b_out):
    del block1_d2c

    hidx = jnp.stack([
        _pad_idx(block0_c2d[0], N_NODE),
        _pad_idx(block0_c2d[1], N_NODE),
        _pad_idx(block0_d2c[0], N_NODE),
        _pad_idx(block0_d2c[1], N_NODE),
        _pad_idx(block1_c2d[0], N_NODE),
        _pad_idx(block1_c2d[1], N_NODE),
    ])
    sg_c2d = _src_gather_idx(block0_c2d[0])
    ds_c2d = _pad_idx(block0_c2d[1], N_NODE)
    sg_d2c = _src_gather_idx(block0_d2c[0])
    ds_d2c = _pad_idx(block0_d2c[1], N_NODE)
    sg_b1 = _src_gather_idx(block1_c2d[0])
    ds_b1 = _pad_idx(block1_c2d[1], N_NODE)

    b_drug2 = b_drug.reshape(1, CD)
    b_cell2 = b_cell.reshape(1, CD)
    b_expr2 = b_expr.reshape(1, -1)
    b1_d2c2 = b1_d2c.reshape(1, CD)
    b1_c2d2 = b1_c2d.reshape(1, CD)
    b2_c2d2 = b2_c2d.reshape(1, CD)
    b_mid2 = b_mid.reshape(1, -1)
    b_out2 = b_out.reshape(1, 1)

    z128 = jnp.zeros((128, 128), jnp.float32)

    cell_enc = _encoder(cell_features_in_network, W_cell, b_cell2, 1000)
    drug_enc = _encoder(drug_features, W_drug, b_drug2, 1000)
    deg = _degrees(hidx, z128)
    d0, d1t, d2, d3, d4, d5t = (deg[i] for i in range(6))

    t_cell = _scale(cell_enc, d0, 2000)
    t_drug = _scale(drug_enc, d2, 2000)

    m_cell = _segsum(t_drug.reshape(NCHUNK * N_NODE, CHUNK), sg_d2c, ds_d2c,
                     z128)
    m_drug = _segsum(t_cell.reshape(NCHUNK * N_NODE, CHUNK), sg_c2d, ds_c2d,
                     z128)

    m1_sel, enc_sel, d1_sel, d5_sel = _gather1(
        m_drug.reshape(NCHUNK * N_NODE, CHUNK), drug_enc, d1t, d5t,
        drug_index)

    t_h1 = _h1cell(m_cell, W1_d2c, b1_d2c2, cell_enc, d3, d4, 1000)

    m2 = _segsum(t_h1.reshape(NCHUNK * N_NODE, CHUNK), sg_b1, ds_b1, z128)
    m2_sel = _gather2(m2.reshape(NCHUNK * N_NODE, CHUNK), drug_index)

    expr_enc = _expr(cell_features, W_expr, b_expr2, 512)
    out = _head(expr_enc, m1_sel, enc_sel, d1_sel, d5_sel, m2_sel,
                W1_c2d, b1_c2d2, W2_c2d, b2_c2d2, W_mid, b_mid2, W_out,
                b_out2)
    return out


# speedup vs baseline: 3.7669x; 1.0185x over previous
import functools

import jax
import jax.numpy as jnp
from jax import lax
from jax.experimental import pallas as pl
from jax.experimental.pallas import tpu as pltpu
from jax.experimental.pallas import tpu_sc as plsc

N_NODE = 10000
N_PAD = 10016
E = 38000
CD = 512
CHUNK = 128
NCHUNK = CD // CHUNK
NTILE = 16
EPT = E // NTILE
NB = 19
B = 1024
_ROWS = 624
_ZTAIL = N_PAD - NTILE * _ROWS
_DTAIL = N_NODE - NTILE * _ROWS


def _lrelu(x):
    return jnp.where(x >= 0, x, 0.01 * x)


def _mesh():
    return plsc.VectorSubcoreMesh(core_axis_name="c", subcore_axis_name="s")


def _zero_slices(zbuf, acc, s, tail):
    for p, sz in ((0, 128), (1, 128), (2, 128), (3, 128), (4, 112)):
        pltpu.sync_copy(zbuf.at[pl.ds(0, sz)],
                        acc.at[pl.ds(s * _ROWS + p * 128, sz)])

    @pl.when(s == NTILE - 1)
    def _():
        pltpu.sync_copy(zbuf.at[pl.ds(0, tail)],
                        acc.at[pl.ds(NTILE * _ROWS, tail)])


def _dump_slices(acc, out_slice_fn, s):
    pltpu.sync_copy(acc.at[pl.ds(s * _ROWS, _ROWS)],
                    out_slice_fn(s * _ROWS, _ROWS))

    @pl.when(s == NTILE - 1)
    def _():
        pltpu.sync_copy(acc.at[pl.ds(NTILE * _ROWS, _DTAIL)],
                        out_slice_fn(NTILE * _ROWS, _DTAIL))


def _fill_const(buf, n, val):
    def _row(i, _):
        for q in range(buf.shape[1] // 16):
            buf[i, pl.ds(q * 16, 16)] = jnp.full((16,), val, jnp.float32)
        return 0

    lax.fori_loop(0, n, _row, 0)


def _deg_body(hidx, z128, deg, a128, idx, o128, sem):
    c = lax.axis_index("c")
    s = lax.axis_index("s")
    _fill_const(o128, 128, 1.0)

    def _round(h, out_idx):
        _zero_slices(z128, a128, s, _ZTAIL)
        plsc.subcore_barrier()
        pltpu.sync_copy(hidx.at[h, s], idx)
        descs = [pltpu.async_copy(o128, a128.at[idx.at[j]], sem, add=True)
                 for j in range(NB)]
        for dsc in descs:
            dsc.wait()
        plsc.subcore_barrier()
        _dump_slices(a128, lambda o, n: deg.at[out_idx, pl.ds(o, n)], s)
        plsc.subcore_barrier()

    _round(3 * c, 3 * c)
    _round(1 + 4 * c, 1 + 4 * c)
    _round(2 + 2 * c, 2 + 2 * c)


def _degrees(hidx, z128):
    k = pl.kernel(
        _deg_body,
        out_type=jax.ShapeDtypeStruct((6, N_NODE, 128), jnp.float32),
        mesh=_mesh(),
        scratch_types=[
            pltpu.VMEM_SHARED((N_PAD, 128), jnp.float32),
            pltpu.VMEM((NB, 128), jnp.int32),
            pltpu.VMEM((128, 128), jnp.float32),
            pltpu.SemaphoreType.DMA,
        ],
    )
    return k(hidx, z128)


def _segsum_body(tflat, src_g, dst_s, zbuf, out, acc, idxs, idxd, r0, r1,
                 g0, g1, s0, s1):
    c = lax.axis_index("c")
    s = lax.axis_index("s")
    pltpu.sync_copy(dst_s.at[s], idxd)
    bufs, gsems, ssems = (r0, r1), (g0, g1), (s0, s1)

    for cc in range(2):
        c2 = 2 * c + cc
        _zero_slices(zbuf, acc, s, _ZTAIL)
        plsc.subcore_barrier()
        pltpu.sync_copy(src_g.at[c2, s], idxs)
        gd = [None, None]
        sd = [None, None]
        for j in range(NB + 1):
            if j < NB:
                b = j % 2
                if sd[b] is not None:
                    sd[b].wait()
                gd[b] = pltpu.async_copy(tflat.at[idxs.at[j]], bufs[b],
                                         gsems[b])
            if j >= 1:
                bb = (j - 1) % 2
                gd[bb].wait()
                sd[bb] = pltpu.async_copy(bufs[bb], acc.at[idxd.at[j - 1]],
                                          ssems[bb], add=True)
        sd[(NB - 1) % 2].wait()
        sd[NB % 2].wait()
        plsc.subcore_barrier()
        _dump_slices(acc, lambda o, n: out.at[c2, pl.ds(o, n)], s)
        plsc.subcore_barrier()


def _segsum(tflat, src_g, dst_s, z128):
    k = pl.kernel(
        _segsum_body,
        out_type=jax.ShapeDtypeStruct((NCHUNK, N_NODE, CHUNK), jnp.float32),
        mesh=_mesh(),
        scratch_types=[
            pltpu.VMEM_SHARED((N_PAD, CHUNK), jnp.float32),
            pltpu.VMEM((NB, 128), jnp.int32),
            pltpu.VMEM((NB, 128), jnp.int32),
            pltpu.VMEM((128, CHUNK), jnp.float32),
            pltpu.VMEM((128, CHUNK), jnp.float32),
            pltpu.SemaphoreType.DMA,
            pltpu.SemaphoreType.DMA,
            pltpu.SemaphoreType.DMA,
            pltpu.SemaphoreType.DMA,
        ],
    )
    return k(tflat, src_g, dst_s, z128)


def _segsum_sel_body(tflat, src_g, dst_s, zbuf, dix, out, acc, idxs, idxd,
                     r0, r1, selv, selrows, g0, g1, s0, s1):
    c = lax.axis_index("c")
    s = lax.axis_index("s")
    pltpu.sync_copy(dst_s.at[s], idxd)
    pltpu.sync_copy(dix.at[pl.ds(s * (B // NTILE), B // NTILE)], selv)
    bufs, gsems, ssems = (r0, r1), (g0, g1), (s0, s1)

    for cc in range(2):
        c2 = 2 * c + cc
        _zero_slices(zbuf, acc, s, _ZTAIL)
        plsc.subcore_barrier()
        pltpu.sync_copy(src_g.at[c2, s], idxs)
        gd = [None, None]
        sd = [None, None]
        for j in range(NB + 1):
            if j < NB:
                b = j % 2
                if sd[b] is not None:
                    sd[b].wait()
                gd[b] = pltpu.async_copy(tflat.at[idxs.at[j]], bufs[b],
                                         gsems[b])
            if j >= 1:
                bb = (j - 1) % 2
                gd[bb].wait()
                sd[bb] = pltpu.async_copy(bufs[bb], acc.at[idxd.at[j - 1]],
                                          ssems[bb], add=True)
        sd[(NB - 1) % 2].wait()
        sd[NB % 2].wait()
        plsc.subcore_barrier()
        pltpu.async_copy(acc.at[selv], selrows, g0).wait()
        pltpu.sync_copy(selrows,
                        out.at[c2, pl.ds(s * (B // NTILE), B // NTILE)])
        plsc.subcore_barrier()


def _segsum_sel(tflat, src_g, dst_s, z128, dix):
    k = pl.kernel(
        _segsum_sel_body,
        out_type=jax.ShapeDtypeStruct((NCHUNK, B, CHUNK), jnp.float32),
        mesh=_mesh(),
        scratch_types=[
            pltpu.VMEM_SHARED((N_PAD, CHUNK), jnp.float32),
            pltpu.VMEM((NB, 128), jnp.int32),
            pltpu.VMEM((NB, 128), jnp.int32),
            pltpu.VMEM((128, CHUNK), jnp.float32),
            pltpu.VMEM((128, CHUNK), jnp.float32),
            pltpu.VMEM((B // NTILE,), jnp.int32),
            pltpu.VMEM((B // NTILE, CHUNK), jnp.float32),
            pltpu.SemaphoreType.DMA,
            pltpu.SemaphoreType.DMA,
            pltpu.SemaphoreType.DMA,
            pltpu.SemaphoreType.DMA,
        ],
    )
    return k(tflat, src_g, dst_s, z128, dix)


def _gather1_body(denc, deg1t, deg5t, dix, enc_sel, d1_sel, d5_sel, idxv,
                  r128, r512, sem):
    c = lax.axis_index("c")
    s = lax.axis_index("s")
    base = (s * 2 + c) * 32
    pltpu.sync_copy(dix.at[pl.ds(base, 32)], idxv)
    pltpu.async_copy(denc.at[idxv], r512, sem).wait()
    pltpu.sync_copy(r512, enc_sel.at[pl.ds(base, 32)])
    pltpu.async_copy(deg1t.at[idxv], r128, sem).wait()
    pltpu.sync_copy(r128, d1_sel.at[pl.ds(base, 32)])
    pltpu.async_copy(deg5t.at[idxv], r128, sem).wait()
    pltpu.sync_copy(r128, d5_sel.at[pl.ds(base, 32)])


def _gather1(denc, deg1t, deg5t, dix):
    k = pl.kernel(
        _gather1_body,
        out_type=(
            jax.ShapeDtypeStruct((B, CD), jnp.float32),
            jax.ShapeDtypeStruct((B, 128), jnp.float32),
            jax.ShapeDtypeStruct((B, 128), jnp.float32),
        ),
        mesh=_mesh(),
        scratch_types=[
            pltpu.VMEM((32,), jnp.int32),
            pltpu.VMEM((32, CHUNK), jnp.float32),
            pltpu.VMEM((32, CD), jnp.float32),
            pltpu.SemaphoreType.DMA,
        ],
    )
    return k(denc, deg1t, deg5t, dix)


def _enc_body(x, w, b, enc):
    a = jnp.dot(x[...], w[...], preferred_element_type=jnp.float32)
    enc[...] = _lrelu(a + b[...])


def _encoder(x, w, b, bm):
    n, kdim = x.shape
    return pl.pallas_call(
        _enc_body,
        grid=(n // bm,),
        in_specs=[
            pl.BlockSpec((bm, kdim), lambda m: (m, 0)),
            pl.BlockSpec((kdim, CD), lambda m: (0, 0)),
            pl.BlockSpec((1, CD), lambda m: (0, 0)),
        ],
        out_specs=pl.BlockSpec((bm, CD), lambda m: (m, 0)),
        out_shape=jax.ShapeDtypeStruct((n, CD), jnp.float32),
    )(x, w, b)


def _scale_body(enc, deg, tout):
    sc = lax.rsqrt(jnp.maximum(deg[:, 0:1], 1.0))
    t = enc[...] * sc
    for ch in range(NCHUNK):
        tout[ch] = t[:, ch * CHUNK:(ch + 1) * CHUNK]


def _scale(enc, deg, bm):
    n = enc.shape[0]
    return pl.pallas_call(
        _scale_body,
        grid=(n // bm,),
        in_specs=[
            pl.BlockSpec((bm, CD), lambda m: (m, 0)),
            pl.BlockSpec((bm, 128), lambda m: (m, 0)),
        ],
        out_specs=pl.BlockSpec((NCHUNK, bm, CHUNK), lambda m: (0, m, 0)),
        out_shape=jax.ShapeDtypeStruct((NCHUNK, n, CHUNK), jnp.float32),
    )(enc, deg)


def _h1cell_body(m, w, b, enc, dgi, dgo, tout, acc):
    k = pl.program_id(1)

    @pl.when(k == 0)
    def _():
        acc[...] = jnp.zeros_like(acc)

    acc[...] += jnp.dot(m[0], w[...], preferred_element_type=jnp.float32)

    @pl.when(k == NCHUNK - 1)
    def _():
        si = lax.rsqrt(jnp.maximum(dgi[:, 0:1], 1.0))
        so = lax.rsqrt(jnp.maximum(dgo[:, 0:1], 1.0))
        h = _lrelu(acc[...] * si + b[...] + 0.5 * enc[...]) * so
        for ch in range(NCHUNK):
            tout[ch] = h[:, ch * CHUNK:(ch + 1) * CHUNK]


def _h1cell(m_cell, w, b, enc, dgi, dgo, bm):
    n = enc.shape[0]
    grid = (n // bm, NCHUNK)
    return pl.pallas_call(
        _h1cell_body,
        grid=grid,
        in_specs=[
            pl.BlockSpec((1, bm, CHUNK), lambda m, k: (k, m, 0)),
            pl.BlockSpec((CHUNK, CD), lambda m, k: (k, 0)),
            pl.BlockSpec((1, CD), lambda m, k: (0, 0)),
            pl.BlockSpec((bm, CD), lambda m, k: (m, 0)),
            pl.BlockSpec((bm, 128), lambda m, k: (m, 0)),
            pl.BlockSpec((bm, 128), lambda m, k: (m, 0)),
        ],
        out_specs=pl.BlockSpec((NCHUNK, bm, CHUNK), lambda m, k: (0, m, 0)),
        out_shape=jax.ShapeDtypeStruct((NCHUNK, n, CHUNK), jnp.float32),
        scratch_shapes=[pltpu.VMEM((bm, CD), jnp.float32)],
    )(m_cell, w, b, enc, dgi, dgo)


def _expr_body(x, w, b, out):
    out[...] = _lrelu(
        jnp.dot(x[...], w[...], preferred_element_type=jnp.float32) + b[...])


def _expr(x, w, b, bm):
    n, kdim = x.shape
    ee = w.shape[1]
    return pl.pallas_call(
        _expr_body,
        grid=(n // bm,),
        in_specs=[
            pl.BlockSpec((bm, kdim), lambda m: (m, 0)),
            pl.BlockSpec((kdim, ee), lambda m: (0, 0)),
            pl.BlockSpec((1, ee), lambda m: (0, 0)),
        ],
        out_specs=pl.BlockSpec((bm, ee), lambda m: (m, 0)),
        out_shape=jax.ShapeDtypeStruct((n, ee), jnp.float32),
    )(x, w, b)


def _head_body(expr, m1, enc1, d1, d5, m2, w1, b1, w2, b2, wm, bm_, wo, bo,
               out):
    p1 = jnp.zeros((B, CD), jnp.float32)
    p2 = jnp.zeros((B, CD), jnp.float32)
    for ch in range(NCHUNK):
        wch = w1[pl.ds(ch * CHUNK, CHUNK), :]
        p1 = p1 + jnp.dot(m1[ch], wch, preferred_element_type=jnp.float32)
        wch2 = w2[pl.ds(ch * CHUNK, CHUNK), :]
        p2 = p2 + jnp.dot(m2[ch], wch2, preferred_element_type=jnp.float32)
    s1 = lax.rsqrt(jnp.maximum(d1[:, 0:1], 1.0))
    s5 = lax.rsqrt(jnp.maximum(d5[:, 0:1], 1.0))
    h1 = _lrelu(p1 * s1 + b1[...] + 0.5 * enc1[...])
    h2 = _lrelu(p2 * s5 + b2[...] + 0.5 * h1)
    ee = expr.shape[1]
    mid = _lrelu(
        jnp.dot(expr[...], wm[pl.ds(0, ee), :],
                preferred_element_type=jnp.float32)
        + jnp.dot(h2, wm[pl.ds(ee, CD), :],
                  preferred_element_type=jnp.float32)
        + bm_[...])
    out[...] = jnp.dot(mid, wo[...], preferred_element_type=jnp.float32) \
        + bo[...]


def _head(expr, m1, enc1, d1, d5, m2, w1, b1, w2, b2, wm, bmid, wo, bo):
    ee = expr.shape[1]
    mids = wm.shape[1]
    full = lambda *shape: pl.BlockSpec(shape, lambda: tuple(0 for _ in shape))
    return pl.pallas_call(
        _head_body,
        grid=(),
        in_specs=[
            full(B, ee),
            full(NCHUNK, B, CHUNK),
            full(B, CD),
            full(B, 128),
            full(B, 128),
            full(NCHUNK, B, CHUNK),
            full(CD, CD),
            full(1, CD),
            full(CD, CD),
            full(1, CD),
            full(ee + CD, mids),
            full(1, mids),
            full(mids, 1),
            full(1, 1),
        ],
        out_specs=full(B, 1),
        out_shape=jax.ShapeDtypeStruct((B, 1), jnp.float32),
    )(expr, m1, enc1, d1, d5, m2, w1, b1, w2, b2, wm, bmid, wo, bo)


def _pad_idx(a, fill):
    a = a.reshape(NTILE, EPT)
    a = jnp.pad(a, ((0, 0), (0, NB * 128 - EPT)), constant_values=fill)
    return a.reshape(NTILE, NB, 128)


def _src_gather_idx(src):
    base = _pad_idx(src, 0)
    offs = (jnp.arange(NCHUNK, dtype=jnp.int32) * N_NODE)[:, None, None, None]
    return base[None] + offs


def kernel(drug_features, cell_features_in_network, cell_features, drug_index,
           block0_d2c, block0_c2d, block1_d2c, block1_c2d, W_drug, b_drug,
           W_cell, b_cell, W_expr, b_expr, W1_d2c, b1_d2c, W1_c2d, b1_c2d,
           W2_c2d, b2_c2d, W_mid, b_mid, W_out, b_out):
    del block1_d2c

    hidx = jnp.stack([
        _pad_idx(block0_c2d[0], N_NODE),
        _pad_idx(block0_c2d[1], N_NODE),
        _pad_idx(block0_d2c[0], N_NODE),
        _pad_idx(block0_d2c[1], N_NODE),
        _pad_idx(block1_c2d[0], N_NODE),
        _pad_idx(block1_c2d[1], N_NODE),
    ])
    sg_c2d = _src_gather_idx(block0_c2d[0])
    ds_c2d = _pad_idx(block0_c2d[1], N_NODE)
    sg_d2c = _src_gather_idx(block0_d2c[0])
    ds_d2c = _pad_idx(block0_d2c[1], N_NODE)
    sg_b1 = _src_gather_idx(block1_c2d[0])
    ds_b1 = _pad_idx(block1_c2d[1], N_NODE)

    b_drug2 = b_drug.reshape(1, CD)
    b_cell2 = b_cell.reshape(1, CD)
    b_expr2 = b_expr.reshape(1, -1)
    b1_d2c2 = b1_d2c.reshape(1, CD)
    b1_c2d2 = b1_c2d.reshape(1, CD)
    b2_c2d2 = b2_c2d.reshape(1, CD)
    b_mid2 = b_mid.reshape(1, -1)
    b_out2 = b_out.reshape(1, 1)

    z128 = jnp.zeros((128, 128), jnp.float32)

    cell_enc = _encoder(cell_features_in_network, W_cell, b_cell2, 1000)
    drug_enc = _encoder(drug_features, W_drug, b_drug2, 1000)
    deg = _degrees(hidx, z128)
    d0, d1t, d2, d3, d4, d5t = (deg[i] for i in range(6))

    t_cell = _scale(cell_enc, d0, 2000)
    t_drug = _scale(drug_enc, d2, 2000)

    m_cell = _segsum(t_drug.reshape(NCHUNK * N_NODE, CHUNK), sg_d2c, ds_d2c,
                     z128)
    m1_sel = _segsum_sel(t_cell.reshape(NCHUNK * N_NODE, CHUNK), sg_c2d,
                         ds_c2d, z128, drug_index)

    enc_sel, d1_sel, d5_sel = _gather1(drug_enc, d1t, d5t, drug_index)

    t_h1 = _h1cell(m_cell, W1_d2c, b1_d2c2, cell_enc, d3, d4, 1000)

    m2_sel = _segsum_sel(t_h1.reshape(NCHUNK * N_NODE, CHUNK), sg_b1, ds_b1,
                         z128, drug_index)

    expr_enc = _expr(cell_features, W_expr, b_expr2, 512)
    out = _head(expr_enc, m1_sel, enc_sel, d1_sel, d5_sel, m2_sel,
                W1_c2d, b1_c2d2, W2_c2d, b2_c2d2, W_mid, b_mid2, W_out,
                b_out2)
    return out


# speedup vs baseline: 3.8398x; 1.0193x over previous
import functools

import jax
import jax.numpy as jnp
from jax import lax
from jax.experimental import pallas as pl
from jax.experimental.pallas import tpu as pltpu
from jax.experimental.pallas import tpu_sc as plsc

N_NODE = 10000
N_PAD = 10016
E = 38000
CD = 512
CHUNK = 128
NCHUNK = CD // CHUNK
NTILE = 16
EPT = E // NTILE
NB = 19
EB = 64
NBB = 38
NBUF = 3
B = 1024
_ROWS = 624
_ZTAIL = N_PAD - NTILE * _ROWS
_DTAIL = N_NODE - NTILE * _ROWS


def _lrelu(x):
    return jnp.where(x >= 0, x, 0.01 * x)


def _mesh():
    return plsc.VectorSubcoreMesh(core_axis_name="c", subcore_axis_name="s")


def _zero_slices(zbuf, acc, s, tail):
    for p, sz in ((0, 128), (1, 128), (2, 128), (3, 128), (4, 112)):
        pltpu.sync_copy(zbuf.at[pl.ds(0, sz)],
                        acc.at[pl.ds(s * _ROWS + p * 128, sz)])

    @pl.when(s == NTILE - 1)
    def _():
        pltpu.sync_copy(zbuf.at[pl.ds(0, tail)],
                        acc.at[pl.ds(NTILE * _ROWS, tail)])


def _dump_slices(acc, out_slice_fn, s):
    pltpu.sync_copy(acc.at[pl.ds(s * _ROWS, _ROWS)],
                    out_slice_fn(s * _ROWS, _ROWS))

    @pl.when(s == NTILE - 1)
    def _():
        pltpu.sync_copy(acc.at[pl.ds(NTILE * _ROWS, _DTAIL)],
                        out_slice_fn(NTILE * _ROWS, _DTAIL))


def _fill_const(buf, n, val):
    def _row(i, _):
        for q in range(buf.shape[1] // 16):
            buf[i, pl.ds(q * 16, 16)] = jnp.full((16,), val, jnp.float32)
        return 0

    lax.fori_loop(0, n, _row, 0)


def _deg_body(hidx, z128, deg, a128, idx, o128, sem):
    c = lax.axis_index("c")
    s = lax.axis_index("s")
    _fill_const(o128, 128, 1.0)

    def _round(h, out_idx):
        _zero_slices(z128, a128, s, _ZTAIL)
        plsc.subcore_barrier()
        pltpu.sync_copy(hidx.at[h, s], idx)
        descs = [pltpu.async_copy(o128, a128.at[idx.at[j]], sem, add=True)
                 for j in range(NB)]
        for dsc in descs:
            dsc.wait()
        plsc.subcore_barrier()
        _dump_slices(a128, lambda o, n: deg.at[out_idx, pl.ds(o, n)], s)
        plsc.subcore_barrier()

    _round(3 * c, 3 * c)
    _round(1 + 4 * c, 1 + 4 * c)
    _round(2 + 2 * c, 2 + 2 * c)


def _degrees(hidx, z128):
    k = pl.kernel(
        _deg_body,
        out_type=jax.ShapeDtypeStruct((6, N_NODE, 128), jnp.float32),
        mesh=_mesh(),
        scratch_types=[
            pltpu.VMEM_SHARED((N_PAD, 128), jnp.float32),
            pltpu.VMEM((NB, 128), jnp.int32),
            pltpu.VMEM((128, 128), jnp.float32),
            pltpu.SemaphoreType.DMA,
        ],
    )
    return k(hidx, z128)


def _segsum_round(tflat, src_g, dst_s, acc, idxs, idxd, bufs, gsems, ssems,
                  c2, s):
    pltpu.sync_copy(src_g.at[c2, s], idxs)
    gd = [None] * NBUF
    sd = [None] * NBUF
    for j in range(NBB + 1):
        if j < NBB:
            b = j % NBUF
            if sd[b] is not None:
                sd[b].wait()
            gd[b] = pltpu.async_copy(tflat.at[idxs.at[j]], bufs[b], gsems[b])
        if j >= 1:
            bb = (j - 1) % NBUF
            gd[bb].wait()
            sd[bb] = pltpu.async_copy(bufs[bb], acc.at[idxd.at[j - 1]],
                                      ssems[bb], add=True)
    for i in range(NBUF):
        sd[(NBB - NBUF + i) % NBUF].wait()


def _segsum_body(tflat, src_g, dst_s, zbuf, out, acc, idxs, idxd, r0, r1,
                 r2, g0, g1, g2, s0, s1, s2):
    c = lax.axis_index("c")
    s = lax.axis_index("s")
    pltpu.sync_copy(dst_s.at[s], idxd)
    bufs, gsems, ssems = (r0, r1, r2), (g0, g1, g2), (s0, s1, s2)

    for cc in range(2):
        c2 = 2 * c + cc
        _zero_slices(zbuf, acc, s, _ZTAIL)
        plsc.subcore_barrier()
        _segsum_round(tflat, src_g, dst_s, acc, idxs, idxd, bufs, gsems,
                      ssems, c2, s)
        plsc.subcore_barrier()
        _dump_slices(acc, lambda o, n: out.at[c2, pl.ds(o, n)], s)
        plsc.subcore_barrier()


def _segsum(tflat, src_g, dst_s, z128):
    k = pl.kernel(
        _segsum_body,
        out_type=jax.ShapeDtypeStruct((NCHUNK, N_NODE, CHUNK), jnp.float32),
        mesh=_mesh(),
        scratch_types=[
            pltpu.VMEM_SHARED((N_PAD, CHUNK), jnp.float32),
            pltpu.VMEM((NBB, EB), jnp.int32),
            pltpu.VMEM((NBB, EB), jnp.int32),
            pltpu.VMEM((EB, CHUNK), jnp.float32),
            pltpu.VMEM((EB, CHUNK), jnp.float32),
            pltpu.VMEM((EB, CHUNK), jnp.float32),
        ] + [pltpu.SemaphoreType.DMA] * 6,
    )
    return k(tflat, src_g, dst_s, z128)


def _segsum_sel_body(tflat, src_g, dst_s, zbuf, dix, out, acc, idxs, idxd,
                     r0, r1, r2, selv, selrows, g0, g1, g2, s0, s1, s2):
    c = lax.axis_index("c")
    s = lax.axis_index("s")
    pltpu.sync_copy(dst_s.at[s], idxd)
    pltpu.sync_copy(dix.at[pl.ds(s * (B // NTILE), B // NTILE)], selv)
    bufs, gsems, ssems = (r0, r1, r2), (g0, g1, g2), (s0, s1, s2)

    for cc in range(2):
        c2 = 2 * c + cc
        _zero_slices(zbuf, acc, s, _ZTAIL)
        plsc.subcore_barrier()
        _segsum_round(tflat, src_g, dst_s, acc, idxs, idxd, bufs, gsems,
                      ssems, c2, s)
        plsc.subcore_barrier()
        pltpu.async_copy(acc.at[selv], selrows, g0).wait()
        pltpu.sync_copy(selrows,
                        out.at[c2, pl.ds(s * (B // NTILE), B // NTILE)])
        plsc.subcore_barrier()


def _segsum_sel(tflat, src_g, dst_s, z128, dix):
    k = pl.kernel(
        _segsum_sel_body,
        out_type=jax.ShapeDtypeStruct((NCHUNK, B, CHUNK), jnp.float32),
        mesh=_mesh(),
        scratch_types=[
            pltpu.VMEM_SHARED((N_PAD, CHUNK), jnp.float32),
            pltpu.VMEM((NBB, EB), jnp.int32),
            pltpu.VMEM((NBB, EB), jnp.int32),
            pltpu.VMEM((EB, CHUNK), jnp.float32),
            pltpu.VMEM((EB, CHUNK), jnp.float32),
            pltpu.VMEM((EB, CHUNK), jnp.float32),
            pltpu.VMEM((B // NTILE,), jnp.int32),
            pltpu.VMEM((B // NTILE, CHUNK), jnp.float32),
        ] + [pltpu.SemaphoreType.DMA] * 6,
    )
    return k(tflat, src_g, dst_s, z128, dix)


def _gather1_body(denc, deg1t, deg5t, dix, enc_sel, d1_sel, d5_sel, idxv,
                  r128, r512, sem):
    c = lax.axis_index("c")
    s = lax.axis_index("s")
    base = (s * 2 + c) * 32
    pltpu.sync_copy(dix.at[pl.ds(base, 32)], idxv)
    pltpu.async_copy(denc.at[idxv], r512, sem).wait()
    pltpu.sync_copy(r512, enc_sel.at[pl.ds(base, 32)])
    pltpu.async_copy(deg1t.at[idxv], r128, sem).wait()
    pltpu.sync_copy(r128, d1_sel.at[pl.ds(base, 32)])
    pltpu.async_copy(deg5t.at[idxv], r128, sem).wait()
    pltpu.sync_copy(r128, d5_sel.at[pl.ds(base, 32)])


def _gather1(denc, deg1t, deg5t, dix):
    k = pl.kernel(
        _gather1_body,
        out_type=(
            jax.ShapeDtypeStruct((B, CD), jnp.float32),
            jax.ShapeDtypeStruct((B, 128), jnp.float32),
            jax.ShapeDtypeStruct((B, 128), jnp.float32),
        ),
        mesh=_mesh(),
        scratch_types=[
            pltpu.VMEM((32,), jnp.int32),
            pltpu.VMEM((32, CHUNK), jnp.float32),
            pltpu.VMEM((32, CD), jnp.float32),
            pltpu.SemaphoreType.DMA,
        ],
    )
    return k(denc, deg1t, deg5t, dix)


def _enc_body(x, w, b, enc):
    a = jnp.dot(x[...], w[...], preferred_element_type=jnp.float32)
    enc[...] = _lrelu(a + b[...])


def _encoder(x, w, b, bm):
    n, kdim = x.shape
    return pl.pallas_call(
        _enc_body,
        grid=(n // bm,),
        in_specs=[
            pl.BlockSpec((bm, kdim), lambda m: (m, 0)),
            pl.BlockSpec((kdim, CD), lambda m: (0, 0)),
            pl.BlockSpec((1, CD), lambda m: (0, 0)),
        ],
        out_specs=pl.BlockSpec((bm, CD), lambda m: (m, 0)),
        out_shape=jax.ShapeDtypeStruct((n, CD), jnp.float32),
    )(x, w, b)


def _scale_body(enc, deg, tout):
    sc = lax.rsqrt(jnp.maximum(deg[:, 0:1], 1.0))
    t = enc[...] * sc
    for ch in range(NCHUNK):
        tout[ch] = t[:, ch * CHUNK:(ch + 1) * CHUNK]


def _scale(enc, deg, bm):
    n = enc.shape[0]
    return pl.pallas_call(
        _scale_body,
        grid=(n // bm,),
        in_specs=[
            pl.BlockSpec((bm, CD), lambda m: (m, 0)),
            pl.BlockSpec((bm, 128), lambda m: (m, 0)),
        ],
        out_specs=pl.BlockSpec((NCHUNK, bm, CHUNK), lambda m: (0, m, 0)),
        out_shape=jax.ShapeDtypeStruct((NCHUNK, n, CHUNK), jnp.float32),
    )(enc, deg)


def _h1cell_body(m, w, b, enc, dgi, dgo, tout, acc):
    k = pl.program_id(1)

    @pl.when(k == 0)
    def _():
        acc[...] = jnp.zeros_like(acc)

    acc[...] += jnp.dot(m[0], w[...], preferred_element_type=jnp.float32)

    @pl.when(k == NCHUNK - 1)
    def _():
        si = lax.rsqrt(jnp.maximum(dgi[:, 0:1], 1.0))
        so = lax.rsqrt(jnp.maximum(dgo[:, 0:1], 1.0))
        h = _lrelu(acc[...] * si + b[...] + 0.5 * enc[...]) * so
        for ch in range(NCHUNK):
            tout[ch] = h[:, ch * CHUNK:(ch + 1) * CHUNK]


def _h1cell(m_cell, w, b, enc, dgi, dgo, bm):
    n = enc.shape[0]
    grid = (n // bm, NCHUNK)
    return pl.pallas_call(
        _h1cell_body,
        grid=grid,
        in_specs=[
            pl.BlockSpec((1, bm, CHUNK), lambda m, k: (k, m, 0)),
            pl.BlockSpec((CHUNK, CD), lambda m, k: (k, 0)),
            pl.BlockSpec((1, CD), lambda m, k: (0, 0)),
            pl.BlockSpec((bm, CD), lambda m, k: (m, 0)),
            pl.BlockSpec((bm, 128), lambda m, k: (m, 0)),
            pl.BlockSpec((bm, 128), lambda m, k: (m, 0)),
        ],
        out_specs=pl.BlockSpec((NCHUNK, bm, CHUNK), lambda m, k: (0, m, 0)),
        out_shape=jax.ShapeDtypeStruct((NCHUNK, n, CHUNK), jnp.float32),
        scratch_shapes=[pltpu.VMEM((bm, CD), jnp.float32)],
    )(m_cell, w, b, enc, dgi, dgo)


def _expr_body(x, w, b, out):
    out[...] = _lrelu(
        jnp.dot(x[...], w[...], preferred_element_type=jnp.float32) + b[...])


def _expr(x, w, b, bm):
    n, kdim = x.shape
    ee = w.shape[1]
    return pl.pallas_call(
        _expr_body,
        grid=(n // bm,),
        in_specs=[
            pl.BlockSpec((bm, kdim), lambda m: (m, 0)),
            pl.BlockSpec((kdim, ee), lambda m: (0, 0)),
            pl.BlockSpec((1, ee), lambda m: (0, 0)),
        ],
        out_specs=pl.BlockSpec((bm, ee), lambda m: (m, 0)),
        out_shape=jax.ShapeDtypeStruct((n, ee), jnp.float32),
    )(x, w, b)


def _head_body(expr, m1, enc1, d1, d5, m2, w1, b1, w2, b2, wm, bm_, wo, bo,
               out):
    p1 = jnp.zeros((B, CD), jnp.float32)
    p2 = jnp.zeros((B, CD), jnp.float32)
    for ch in range(NCHUNK):
        wch = w1[pl.ds(ch * CHUNK, CHUNK), :]
        p1 = p1 + jnp.dot(m1[ch], wch, preferred_element_type=jnp.float32)
        wch2 = w2[pl.ds(ch * CHUNK, CHUNK), :]
        p2 = p2 + jnp.dot(m2[ch], wch2, preferred_element_type=jnp.float32)
    s1 = lax.rsqrt(jnp.maximum(d1[:, 0:1], 1.0))
    s5 = lax.rsqrt(jnp.maximum(d5[:, 0:1], 1.0))
    h1 = _lrelu(p1 * s1 + b1[...] + 0.5 * enc1[...])
    h2 = _lrelu(p2 * s5 + b2[...] + 0.5 * h1)
    ee = expr.shape[1]
    mid = _lrelu(
        jnp.dot(expr[...], wm[pl.ds(0, ee), :],
                preferred_element_type=jnp.float32)
        + jnp.dot(h2, wm[pl.ds(ee, CD), :],
                  preferred_element_type=jnp.float32)
        + bm_[...])
    out[...] = jnp.dot(mid, wo[...], preferred_element_type=jnp.float32) \
        + bo[...]


def _head(expr, m1, enc1, d1, d5, m2, w1, b1, w2, b2, wm, bmid, wo, bo):
    ee = expr.shape[1]
    mids = wm.shape[1]
    full = lambda *shape: pl.BlockSpec(shape, lambda: tuple(0 for _ in shape))
    return pl.pallas_call(
        _head_body,
        grid=(),
        in_specs=[
            full(B, ee),
            full(NCHUNK, B, CHUNK),
            full(B, CD),
            full(B, 128),
            full(B, 128),
            full(NCHUNK, B, CHUNK),
            full(CD, CD),
            full(1, CD),
            full(CD, CD),
            full(1, CD),
            full(ee + CD, mids),
            full(1, mids),
            full(mids, 1),
            full(1, 1),
        ],
        out_specs=full(B, 1),
        out_shape=jax.ShapeDtypeStruct((B, 1), jnp.float32),
    )(expr, m1, enc1, d1, d5, m2, w1, b1, w2, b2, wm, bmid, wo, bo)


def _pad_idx(a, fill):
    a = a.reshape(NTILE, EPT)
    a = jnp.pad(a, ((0, 0), (0, NB * 128 - EPT)), constant_values=fill)
    return a.reshape(NTILE, NB, 128)


def _pad_idx64(a, fill):
    a = a.reshape(NTILE, EPT)
    a = jnp.pad(a, ((0, 0), (0, NBB * EB - EPT)), constant_values=fill)
    return a.reshape(NTILE, NBB, EB)


def _src_gather_idx(src):
    base = _pad_idx64(src, 0)
    offs = (jnp.arange(NCHUNK, dtype=jnp.int32) * N_NODE)[:, None, None, None]
    return base[None] + offs


def kernel(drug_features, cell_features_in_network, cell_features, drug_index,
           block0_d2c, block0_c2d, block1_d2c, block1_c2d, W_drug, b_drug,
           W_cell, b_cell, W_expr, b_expr, W1_d2c, b1_d2c, W1_c2d, b1_c2d,
           W2_c2d, b2_c2d, W_mid, b_mid, W_out, b_out):
    del block1_d2c

    hidx = jnp.stack([
        _pad_idx(block0_c2d[0], N_NODE),
        _pad_idx(block0_c2d[1], N_NODE),
        _pad_idx(block0_d2c[0], N_NODE),
        _pad_idx(block0_d2c[1], N_NODE),
        _pad_idx(block1_c2d[0], N_NODE),
        _pad_idx(block1_c2d[1], N_NODE),
    ])
    sg_c2d = _src_gather_idx(block0_c2d[0])
    ds_c2d = _pad_idx64(block0_c2d[1], N_NODE)
    sg_d2c = _src_gather_idx(block0_d2c[0])
    ds_d2c = _pad_idx64(block0_d2c[1], N_NODE)
    sg_b1 = _src_gather_idx(block1_c2d[0])
    ds_b1 = _pad_idx64(block1_c2d[1], N_NODE)

    b_drug2 = b_drug.reshape(1, CD)
    b_cell2 = b_cell.reshape(1, CD)
    b_expr2 = b_expr.reshape(1, -1)
    b1_d2c2 = b1_d2c.reshape(1, CD)
    b1_c2d2 = b1_c2d.reshape(1, CD)
    b2_c2d2 = b2_c2d.reshape(1, CD)
    b_mid2 = b_mid.reshape(1, -1)
    b_out2 = b_out.reshape(1, 1)

    z128 = jnp.zeros((128, 128), jnp.float32)

    cell_enc = _encoder(cell_features_in_network, W_cell, b_cell2, 1000)
    drug_enc = _encoder(drug_features, W_drug, b_drug2, 1000)
    deg = _degrees(hidx, z128)
    d0, d1t, d2, d3, d4, d5t = (deg[i] for i in range(6))

    t_cell = _scale(cell_enc, d0, 2000)
    t_drug = _scale(drug_enc, d2, 2000)

    m_cell = _segsum(t_drug.reshape(NCHUNK * N_NODE, CHUNK), sg_d2c, ds_d2c,
                     z128)
    m1_sel = _segsum_sel(t_cell.reshape(NCHUNK * N_NODE, CHUNK), sg_c2d,
                         ds_c2d, z128, drug_index)

    enc_sel, d1_sel, d5_sel = _gather1(drug_enc, d1t, d5t, drug_index)

    t_h1 = _h1cell(m_cell, W1_d2c, b1_d2c2, cell_enc, d3, d4, 1000)

    m2_sel = _segsum_sel(t_h1.reshape(NCHUNK * N_NODE, CHUNK), sg_b1, ds_b1,
                         z128, drug_index)

    expr_enc = _expr(cell_features, W_expr, b_expr2, 512)
    out = _head(expr_enc, m1_sel, enc_sel, d1_sel, d5_sel, m2_sel,
                W1_c2d, b1_c2d2, W2_c2d, b2_c2d2, W_mid, b_mid2, W_out,
                b_out2)
    return out


# speedup vs baseline: 3.8452x; 1.0014x over previous
import jax
import jax.numpy as jnp
from jax import lax
from jax.experimental import pallas as pl
from jax.experimental.pallas import tpu as pltpu
from jax.experimental.pallas import tpu_sc as plsc

N_NODE = 10000
N_PAD = 10016
E = 38000
CD = 512
CHUNK = 128
NCHUNK = CD // CHUNK
NTILE = 16
EPT = E // NTILE
NB = 19
EB = 64
NBB = 38
NBUF = 3
B = 1024
_ROWS = 624
_ZTAIL = N_PAD - NTILE * _ROWS
_DTAIL = N_NODE - NTILE * _ROWS


def _lrelu(x):
    return jnp.where(x >= 0, x, 0.01 * x)


def _mesh():
    return plsc.VectorSubcoreMesh(core_axis_name="c", subcore_axis_name="s")


def _zero_slices(zbuf, acc, s, tail):
    for p, sz in ((0, 128), (1, 128), (2, 128), (3, 128), (4, 112)):
        pltpu.sync_copy(zbuf.at[pl.ds(0, sz)],
                        acc.at[pl.ds(s * _ROWS + p * 128, sz)])

    @pl.when(s == NTILE - 1)
    def _():
        pltpu.sync_copy(zbuf.at[pl.ds(0, tail)],
                        acc.at[pl.ds(NTILE * _ROWS, tail)])


def _dump_slices(acc, out_slice_fn, s):
    pltpu.sync_copy(acc.at[pl.ds(s * _ROWS, _ROWS)],
                    out_slice_fn(s * _ROWS, _ROWS))

    @pl.when(s == NTILE - 1)
    def _():
        pltpu.sync_copy(acc.at[pl.ds(NTILE * _ROWS, _DTAIL)],
                        out_slice_fn(NTILE * _ROWS, _DTAIL))


def _fill_const(buf, n, val):
    def _row(i, _):
        for q in range(buf.shape[1] // 16):
            buf[i, pl.ds(q * 16, 16)] = jnp.full((16,), val, jnp.float32)
        return 0

    lax.fori_loop(0, n, _row, 0)


def _deg_body(hidx, z128, deg, a128, idx, o128, sem):
    c = lax.axis_index("c")
    s = lax.axis_index("s")
    _fill_const(o128, 128, 1.0)

    def _round(h, out_idx):
        _zero_slices(z128, a128, s, _ZTAIL)
        plsc.subcore_barrier()
        pltpu.sync_copy(hidx.at[h, s], idx)
        descs = [pltpu.async_copy(o128, a128.at[idx.at[j]], sem, add=True)
                 for j in range(NB)]
        for dsc in descs:
            dsc.wait()
        plsc.subcore_barrier()
        _dump_slices(a128, lambda o, n: deg.at[out_idx, pl.ds(o, n)], s)
        plsc.subcore_barrier()

    _round(3 * c, 3 * c)
    _round(1 + 4 * c, 1 + 4 * c)
    _round(2 + 2 * c, 2 + 2 * c)


def _degrees(hidx, z128):
    k = pl.kernel(
        _deg_body,
        out_type=jax.ShapeDtypeStruct((6, N_NODE, 128), jnp.float32),
        mesh=_mesh(),
        scratch_types=[
            pltpu.VMEM_SHARED((N_PAD, 128), jnp.float32),
            pltpu.VMEM((NB, 128), jnp.int32),
            pltpu.VMEM((128, 128), jnp.float32),
            pltpu.SemaphoreType.DMA,
        ],
    )
    return k(hidx, z128)


def _segsum_round(tflat, src_g, dst_s, acc, idxs, idxd, bufs, gsems, ssems,
                  c2, s):
    pltpu.sync_copy(src_g.at[c2, s], idxs)
    gd = [None] * NBUF
    sd = [None] * NBUF
    for j in range(NBB + 1):
        if j < NBB:
            b = j % NBUF
            if sd[b] is not None:
                sd[b].wait()
            gd[b] = pltpu.async_copy(tflat.at[idxs.at[j]], bufs[b], gsems[b])
        if j >= 1:
            bb = (j - 1) % NBUF
            gd[bb].wait()
            sd[bb] = pltpu.async_copy(bufs[bb], acc.at[idxd.at[j - 1]],
                                      ssems[bb], add=True)
    for i in range(NBUF):
        sd[(NBB - NBUF + i) % NBUF].wait()


def _segsum_body(tflat, src_g, dst_s, zbuf, out, acc, idxs, idxd, r0, r1,
                 r2, g0, g1, g2, s0, s1, s2):
    c = lax.axis_index("c")
    s = lax.axis_index("s")
    pltpu.sync_copy(dst_s.at[s], idxd)
    bufs, gsems, ssems = (r0, r1, r2), (g0, g1, g2), (s0, s1, s2)

    for cc in range(2):
        c2 = 2 * c + cc
        _zero_slices(zbuf, acc, s, _ZTAIL)
        plsc.subcore_barrier()
        _segsum_round(tflat, src_g, dst_s, acc, idxs, idxd, bufs, gsems,
                      ssems, c2, s)
        plsc.subcore_barrier()
        _dump_slices(acc, lambda o, n: out.at[c2, pl.ds(o, n)], s)
        plsc.subcore_barrier()


def _segsum(tflat, src_g, dst_s, z128):
    k = pl.kernel(
        _segsum_body,
        out_type=jax.ShapeDtypeStruct((NCHUNK, N_NODE, CHUNK), jnp.float32),
        mesh=_mesh(),
        scratch_types=[
            pltpu.VMEM_SHARED((N_PAD, CHUNK), jnp.float32),
            pltpu.VMEM((NBB, EB), jnp.int32),
            pltpu.VMEM((NBB, EB), jnp.int32),
            pltpu.VMEM((EB, CHUNK), jnp.float32),
            pltpu.VMEM((EB, CHUNK), jnp.float32),
            pltpu.VMEM((EB, CHUNK), jnp.float32),
        ] + [pltpu.SemaphoreType.DMA] * 6,
    )
    return k(tflat, src_g, dst_s, z128)


def _segsum_sel_body(tflat, src_g, dst_s, zbuf, dix, out, acc, idxs, idxd,
                     r0, r1, r2, selv, selrows, g0, g1, g2, s0, s1, s2):
    c = lax.axis_index("c")
    s = lax.axis_index("s")
    pltpu.sync_copy(dst_s.at[s], idxd)
    pltpu.sync_copy(dix.at[pl.ds(s * (B // NTILE), B // NTILE)], selv)
    bufs, gsems, ssems = (r0, r1, r2), (g0, g1, g2), (s0, s1, s2)

    for cc in range(2):
        c2 = 2 * c + cc
        _zero_slices(zbuf, acc, s, _ZTAIL)
        plsc.subcore_barrier()
        _segsum_round(tflat, src_g, dst_s, acc, idxs, idxd, bufs, gsems,
                      ssems, c2, s)
        plsc.subcore_barrier()
        pltpu.async_copy(acc.at[selv], selrows, g0).wait()
        pltpu.sync_copy(selrows,
                        out.at[c2, pl.ds(s * (B // NTILE), B // NTILE)])
        plsc.subcore_barrier()


def _segsum_sel(tflat, src_g, dst_s, z128, dix):
    k = pl.kernel(
        _segsum_sel_body,
        out_type=jax.ShapeDtypeStruct((NCHUNK, B, CHUNK), jnp.float32),
        mesh=_mesh(),
        scratch_types=[
            pltpu.VMEM_SHARED((N_PAD, CHUNK), jnp.float32),
            pltpu.VMEM((NBB, EB), jnp.int32),
            pltpu.VMEM((NBB, EB), jnp.int32),
            pltpu.VMEM((EB, CHUNK), jnp.float32),
            pltpu.VMEM((EB, CHUNK), jnp.float32),
            pltpu.VMEM((EB, CHUNK), jnp.float32),
            pltpu.VMEM((B // NTILE,), jnp.int32),
            pltpu.VMEM((B // NTILE, CHUNK), jnp.float32),
        ] + [pltpu.SemaphoreType.DMA] * 6,
    )
    return k(tflat, src_g, dst_s, z128, dix)


def _gather1_body(denc, deg1t, deg5t, dix, enc_sel, d1_sel, d5_sel, idxv,
                  r128, r512, sem):
    c = lax.axis_index("c")
    s = lax.axis_index("s")
    base = (s * 2 + c) * 32
    pltpu.sync_copy(dix.at[pl.ds(base, 32)], idxv)
    pltpu.async_copy(denc.at[idxv], r512, sem).wait()
    pltpu.sync_copy(r512, enc_sel.at[pl.ds(base, 32)])
    pltpu.async_copy(deg1t.at[idxv], r128, sem).wait()
    pltpu.sync_copy(r128, d1_sel.at[pl.ds(base, 32)])
    pltpu.async_copy(deg5t.at[idxv], r128, sem).wait()
    pltpu.sync_copy(r128, d5_sel.at[pl.ds(base, 32)])


def _gather1(denc, deg1t, deg5t, dix):
    k = pl.kernel(
        _gather1_body,
        out_type=(
            jax.ShapeDtypeStruct((B, CD), jnp.float32),
            jax.ShapeDtypeStruct((B, 128), jnp.float32),
            jax.ShapeDtypeStruct((B, 128), jnp.float32),
        ),
        mesh=_mesh(),
        scratch_types=[
            pltpu.VMEM((32,), jnp.int32),
            pltpu.VMEM((32, CHUNK), jnp.float32),
            pltpu.VMEM((32, CD), jnp.float32),
            pltpu.SemaphoreType.DMA,
        ],
    )
    return k(denc, deg1t, deg5t, dix)


def _enc_body(x, w, b, enc):
    a = jnp.dot(x[...], w[...], preferred_element_type=jnp.float32)
    enc[...] = _lrelu(a + b[...])


def _encoder(x, w, b, bm):
    n, kdim = x.shape
    return pl.pallas_call(
        _enc_body,
        grid=(n // bm,),
        in_specs=[
            pl.BlockSpec((bm, kdim), lambda m: (m, 0)),
            pl.BlockSpec((kdim, CD), lambda m: (0, 0)),
            pl.BlockSpec((1, CD), lambda m: (0, 0)),
        ],
        out_specs=pl.BlockSpec((bm, CD), lambda m: (m, 0)),
        out_shape=jax.ShapeDtypeStruct((n, CD), jnp.float32),
    )(x, w, b)


def _scale_body(enc, deg, tout):
    sc = lax.rsqrt(jnp.maximum(deg[:, 0:1], 1.0))
    t = enc[...] * sc
    for ch in range(NCHUNK):
        tout[ch] = t[:, ch * CHUNK:(ch + 1) * CHUNK]


def _scale(enc, deg, bm):
    n = enc.shape[0]
    return pl.pallas_call(
        _scale_body,
        grid=(n // bm,),
        in_specs=[
            pl.BlockSpec((bm, CD), lambda m: (m, 0)),
            pl.BlockSpec((bm, 128), lambda m: (m, 0)),
        ],
        out_specs=pl.BlockSpec((NCHUNK, bm, CHUNK), lambda m: (0, m, 0)),
        out_shape=jax.ShapeDtypeStruct((NCHUNK, n, CHUNK), jnp.float32),
    )(enc, deg)


def _h1cell_body(m, w, b, enc, dgi, dgo, tout, acc):
    k = pl.program_id(1)

    @pl.when(k == 0)
    def _():
        acc[...] = jnp.zeros_like(acc)

    acc[...] += jnp.dot(m[0], w[...], preferred_element_type=jnp.float32)

    @pl.when(k == NCHUNK - 1)
    def _():
        si = lax.rsqrt(jnp.maximum(dgi[:, 0:1], 1.0))
        so = lax.rsqrt(jnp.maximum(dgo[:, 0:1], 1.0))
        h = _lrelu(acc[...] * si + b[...] + 0.5 * enc[...]) * so
        for ch in range(NCHUNK):
            tout[ch] = h[:, ch * CHUNK:(ch + 1) * CHUNK]


def _h1cell(m_cell, w, b, enc, dgi, dgo, bm):
    n = enc.shape[0]
    grid = (n // bm, NCHUNK)
    return pl.pallas_call(
        _h1cell_body,
        grid=grid,
        in_specs=[
            pl.BlockSpec((1, bm, CHUNK), lambda m, k: (k, m, 0)),
            pl.BlockSpec((CHUNK, CD), lambda m, k: (k, 0)),
            pl.BlockSpec((1, CD), lambda m, k: (0, 0)),
            pl.BlockSpec((bm, CD), lambda m, k: (m, 0)),
            pl.BlockSpec((bm, 128), lambda m, k: (m, 0)),
            pl.BlockSpec((bm, 128), lambda m, k: (m, 0)),
        ],
        out_specs=pl.BlockSpec((NCHUNK, bm, CHUNK), lambda m, k: (0, m, 0)),
        out_shape=jax.ShapeDtypeStruct((NCHUNK, n, CHUNK), jnp.float32),
        scratch_shapes=[pltpu.VMEM((bm, CD), jnp.float32)],
    )(m_cell, w, b, enc, dgi, dgo)


def _expr_body(x, w, b, out):
    out[...] = _lrelu(
        jnp.dot(x[...], w[...], preferred_element_type=jnp.float32) + b[...])


def _expr(x, w, b, bm):
    n, kdim = x.shape
    ee = w.shape[1]
    return pl.pallas_call(
        _expr_body,
        grid=(n // bm,),
        in_specs=[
            pl.BlockSpec((bm, kdim), lambda m: (m, 0)),
            pl.BlockSpec((kdim, ee), lambda m: (0, 0)),
            pl.BlockSpec((1, ee), lambda m: (0, 0)),
        ],
        out_specs=pl.BlockSpec((bm, ee), lambda m: (m, 0)),
        out_shape=jax.ShapeDtypeStruct((n, ee), jnp.float32),
    )(x, w, b)


def _head_body(expr, m1, enc1, d1, d5, m2, w1, b1, w2, b2, wm, bm_, wo, bo,
               out):
    p1 = jnp.zeros((B, CD), jnp.float32)
    p2 = jnp.zeros((B, CD), jnp.float32)
    for ch in range(NCHUNK):
        wch = w1[pl.ds(ch * CHUNK, CHUNK), :]
        p1 = p1 + jnp.dot(m1[ch], wch, preferred_element_type=jnp.float32)
        wch2 = w2[pl.ds(ch * CHUNK, CHUNK), :]
        p2 = p2 + jnp.dot(m2[ch], wch2, preferred_element_type=jnp.float32)
    s1 = lax.rsqrt(jnp.maximum(d1[:, 0:1], 1.0))
    s5 = lax.rsqrt(jnp.maximum(d5[:, 0:1], 1.0))
    h1 = _lrelu(p1 * s1 + b1[...] + 0.5 * enc1[...])
    h2 = _lrelu(p2 * s5 + b2[...] + 0.5 * h1)
    ee = expr.shape[1]
    mid = _lrelu(
        jnp.dot(expr[...], wm[pl.ds(0, ee), :],
                preferred_element_type=jnp.float32)
        + jnp.dot(h2, wm[pl.ds(ee, CD), :],
                  preferred_element_type=jnp.float32)
        + bm_[...])
    out[...] = jnp.dot(mid, wo[...], preferred_element_type=jnp.float32) \
        + bo[...]


def _head(expr, m1, enc1, d1, d5, m2, w1, b1, w2, b2, wm, bmid, wo, bo):
    ee = expr.shape[1]
    mids = wm.shape[1]
    full = lambda *shape: pl.BlockSpec(shape, lambda: tuple(0 for _ in shape))
    return pl.pallas_call(
        _head_body,
        grid=(),
        in_specs=[
            full(B, ee),
            full(NCHUNK, B, CHUNK),
            full(B, CD),
            full(B, 128),
            full(B, 128),
            full(NCHUNK, B, CHUNK),
            full(CD, CD),
            full(1, CD),
            full(CD, CD),
            full(1, CD),
            full(ee + CD, mids),
            full(1, mids),
            full(mids, 1),
            full(1, 1),
        ],
        out_specs=full(B, 1),
        out_shape=jax.ShapeDtypeStruct((B, 1), jnp.float32),
    )(expr, m1, enc1, d1, d5, m2, w1, b1, w2, b2, wm, bmid, wo, bo)


def _pad_idx(a, fill):
    a = a.reshape(NTILE, EPT)
    a = jnp.pad(a, ((0, 0), (0, NB * 128 - EPT)), constant_values=fill)
    return a.reshape(NTILE, NB, 128)


def _pad_idx64(a, fill):
    a = a.reshape(NTILE, EPT)
    a = jnp.pad(a, ((0, 0), (0, NBB * EB - EPT)), constant_values=fill)
    return a.reshape(NTILE, NBB, EB)


def _src_gather_idx(src):
    base = _pad_idx64(src, 0)
    offs = (jnp.arange(NCHUNK, dtype=jnp.int32) * N_NODE)[:, None, None, None]
    return base[None] + offs


def kernel(drug_features, cell_features_in_network, cell_features, drug_index,
           block0_d2c, block0_c2d, block1_d2c, block1_c2d, W_drug, b_drug,
           W_cell, b_cell, W_expr, b_expr, W1_d2c, b1_d2c, W1_c2d, b1_c2d,
           W2_c2d, b2_c2d, W_mid, b_mid, W_out, b_out):
    del block1_d2c

    hidx = jnp.stack([
        _pad_idx(block0_c2d[0], N_NODE),
        _pad_idx(block0_c2d[1], N_NODE),
        _pad_idx(block0_d2c[0], N_NODE),
        _pad_idx(block0_d2c[1], N_NODE),
        _pad_idx(block1_c2d[0], N_NODE),
        _pad_idx(block1_c2d[1], N_NODE),
    ])
    sg_c2d = _src_gather_idx(block0_c2d[0])
    ds_c2d = _pad_idx64(block0_c2d[1], N_NODE)
    sg_d2c = _src_gather_idx(block0_d2c[0])
    ds_d2c = _pad_idx64(block0_d2c[1], N_NODE)
    sg_b1 = _src_gather_idx(block1_c2d[0])
    ds_b1 = _pad_idx64(block1_c2d[1], N_NODE)

    b_drug2 = b_drug.reshape(1, CD)
    b_cell2 = b_cell.reshape(1, CD)
    b_expr2 = b_expr.reshape(1, -1)
    b1_d2c2 = b1_d2c.reshape(1, CD)
    b1_c2d2 = b1_c2d.reshape(1, CD)
    b2_c2d2 = b2_c2d.reshape(1, CD)
    b_mid2 = b_mid.reshape(1, -1)
    b_out2 = b_out.reshape(1, 1)

    z128 = jnp.zeros((128, 128), jnp.float32)

    cell_enc = _encoder(cell_features_in_network, W_cell, b_cell2, 1000)
    drug_enc = _encoder(drug_features, W_drug, b_drug2, 1000)
    deg = _degrees(hidx, z128)
    d0, d1t, d2, d3, d4, d5t = (deg[i] for i in range(6))

    t_cell = _scale(cell_enc, d0, 2000)
    t_drug = _scale(drug_enc, d2, 2000)

    m_cell = _segsum(t_drug.reshape(NCHUNK * N_NODE, CHUNK), sg_d2c, ds_d2c,
                     z128)
    m1_sel = _segsum_sel(t_cell.reshape(NCHUNK * N_NODE, CHUNK), sg_c2d,
                         ds_c2d, z128, drug_index)

    enc_sel, d1_sel, d5_sel = _gather1(drug_enc, d1t, d5t, drug_index)

    t_h1 = _h1cell(m_cell, W1_d2c, b1_d2c2, cell_enc, d3, d4, 1000)

    m2_sel = _segsum_sel(t_h1.reshape(NCHUNK * N_NODE, CHUNK), sg_b1, ds_b1,
                         z128, drug_index)

    expr_enc = _expr(cell_features, W_expr, b_expr2, 512)
    out = _head(expr_enc, m1_sel, enc_sel, d1_sel, d5_sel, m2_sel,
                W1_c2d, b1_c2d2, W2_c2d, b2_c2d2, W_mid, b_mid2, W_out,
                b_out2)
    return out
